# Initial kernel scaffold; baseline (speedup 1.0000x reference)
import jax
import jax.numpy as jnp
from jax.experimental import pallas as pl


def kernel(hidden_states, mc_token_ids):
    raise NotImplementedError("write your pallas kernel here")


# speedup vs baseline: 1.1379x; 1.1379x over previous
import functools

import jax
import jax.numpy as jnp
from jax import lax
from jax.experimental import pallas as pl
from jax.experimental.pallas import tpu as pltpu
from jax.experimental.pallas import tpu_sc as plsc

_LANES = 16


@functools.lru_cache(maxsize=None)
def _build_gather(n, s, d):
    assert n % _LANES == 0
    n_workers = n // _LANES
    mesh = plsc.VectorSubcoreMesh(core_axis_name="c", subcore_axis_name="s")
    num_cores = plsc.get_sparse_core_info().num_cores

    @functools.partial(
        pl.kernel,
        mesh=mesh,
        out_type=jax.ShapeDtypeStruct((n, d), jnp.float32),
        scratch_types=[
            pltpu.VMEM((_LANES,), jnp.int32),
            pltpu.VMEM((_LANES, d), jnp.float32),
            pltpu.SemaphoreType.DMA,
        ],
    )
    def gather_kernel(table_hbm, idx_hbm, out_hbm, idx_v, rows_v, sem):
        wid = lax.axis_index("s") * num_cores + lax.axis_index("c")

        @pl.when(wid < n_workers)
        def _():
            base = wid * _LANES
            pltpu.sync_copy(idx_hbm.at[pl.ds(base, _LANES)], idx_v)
            idx_v[...] = idx_v[...] + (lax.iota(jnp.int32, _LANES) + base) * s
            pltpu.async_copy(table_hbm.at[idx_v], rows_v, sem).wait()
            pltpu.sync_copy(rows_v, out_hbm.at[pl.ds(base, _LANES)])

    return gather_kernel


def kernel(hidden_states, mc_token_ids):
    b, c, s, d = hidden_states.shape
    n = b * c
    table = hidden_states.reshape(n * s, d)
    idx = mc_token_ids.reshape(n).astype(jnp.int32)
    out = _build_gather(n, s, d)(table, idx)
    return out.reshape(b, c, d)


# speedup vs baseline: 1.1654x; 1.0242x over previous
import functools

import jax
import jax.numpy as jnp
from jax import lax
from jax.experimental import pallas as pl
from jax.experimental.pallas import tpu as pltpu
from jax.experimental.pallas import tpu_sc as plsc

_LANES = 16


@functools.lru_cache(maxsize=None)
def _build_gather(n, s, d):
    rows_per_worker = 8
    assert n % rows_per_worker == 0
    n_workers = n // rows_per_worker
    mesh = plsc.VectorSubcoreMesh(core_axis_name="c", subcore_axis_name="s")
    num_cores = plsc.get_sparse_core_info().num_cores

    @functools.partial(
        pl.kernel,
        mesh=mesh,
        out_type=jax.ShapeDtypeStruct((n, d), jnp.float32),
        scratch_types=[
            pltpu.VMEM((_LANES,), jnp.int32),
            pltpu.VMEM((rows_per_worker, d), jnp.float32),
            pltpu.SemaphoreType.DMA,
        ],
    )
    def gather_kernel(table_hbm, idx_hbm, out_hbm, idx_v, rows_v, sem):
        wid = lax.axis_index("s") * num_cores + lax.axis_index("c")

        @pl.when(wid < n_workers)
        def _():
            base = wid * rows_per_worker
            pltpu.sync_copy(
                idx_hbm.at[pl.ds(base, rows_per_worker)],
                idx_v.at[pl.ds(0, rows_per_worker)],
            )
            idx_v[...] = idx_v[...] + (lax.iota(jnp.int32, _LANES) + base) * s
            pltpu.async_copy(
                table_hbm.at[idx_v.at[pl.ds(0, rows_per_worker)]], rows_v, sem
            ).wait()
            pltpu.sync_copy(rows_v, out_hbm.at[pl.ds(base, rows_per_worker)])

    return gather_kernel


def kernel(hidden_states, mc_token_ids):
    b, c, s, d = hidden_states.shape
    n = b * c
    table = hidden_states.reshape(n * s, d)
    idx = mc_token_ids.reshape(n).astype(jnp.int32)
    out = _build_gather(n, s, d)(table, idx)
    return out.reshape(b, c, d)


# speedup vs baseline: 1.1924x; 1.0231x over previous
import functools

import jax
import jax.numpy as jnp
from jax import lax
from jax.experimental import pallas as pl
from jax.experimental.pallas import tpu as pltpu
from jax.experimental.pallas import tpu_sc as plsc

_LANES = 16
_ALIGN = 8


@functools.lru_cache(maxsize=None)
def _build_gather(n, s, d):
    n_blocks = n // _ALIGN
    info = plsc.get_sparse_core_info()
    n_workers = info.num_cores * info.num_subcores
    col_chunks = n_workers // n_blocks
    assert d % col_chunks == 0
    dc = d // col_chunks
    mesh = plsc.VectorSubcoreMesh(core_axis_name="c", subcore_axis_name="s")

    @functools.partial(
        pl.kernel,
        mesh=mesh,
        out_type=jax.ShapeDtypeStruct((n, d), jnp.float32),
        scratch_types=[
            pltpu.VMEM((_LANES,), jnp.int32),
            pltpu.VMEM((_ALIGN, dc), jnp.float32),
            pltpu.SemaphoreType.DMA,
        ],
    )
    def gather_kernel(table_hbm, idx_hbm, out_hbm, idx_v, tile_v, sem):
        wid = lax.axis_index("s") * info.num_cores + lax.axis_index("c")
        ib = wid // col_chunks
        cc = wid % col_chunks
        base = ib * _ALIGN
        col = cc * dc
        pltpu.sync_copy(idx_hbm.at[pl.ds(base, _ALIGN)], idx_v.at[pl.ds(0, _ALIGN)])
        idx_v[...] = idx_v[...] + (lax.iota(jnp.int32, _LANES) + base) * s
        pltpu.async_copy(
            table_hbm.at[idx_v.at[pl.ds(0, _ALIGN)], pl.ds(col, dc)], tile_v, sem
        ).wait()
        pltpu.sync_copy(tile_v, out_hbm.at[pl.ds(base, _ALIGN), pl.ds(col, dc)])

    return gather_kernel


def kernel(hidden_states, mc_token_ids):
    b, c, s, d = hidden_states.shape
    n = b * c
    table = hidden_states.reshape(n * s, d)
    idx = mc_token_ids.reshape(n).astype(jnp.int32)
    out = _build_gather(n, s, d)(table, idx)
    return out.reshape(b, c, d)
